# baseline (speedup 1.0000x reference)
import jax
import jax.numpy as jnp
from jax import lax
from jax.experimental import pallas as pl
from jax.experimental.pallas import tpu as pltpu
from jax.experimental.pallas import tpu_sc as plsc

_NUM_PARTS = 100
_NUM_K = 9999
_OUT = 64
_B = 4096
_ROWS = _B * _NUM_PARTS
_V = (_NUM_K + 1) * _NUM_PARTS

_NC = 2
_NS = 16
_NW = _NC * _NS
_CH = 128
_RPW = _ROWS // _NW
_CPW = _RPW // _CH


_PK = 4096
_PSPLIT = 52
_SP_A = 16
_SP_B0 = 15
_BN = 32768
_NB = (_V + _BN - 1) // _BN


def _proj_body(x_ref, w_ref, o_ref):
    y = lax.dot_general(x_ref[...], w_ref[...], (((0,), (1,)), ((), ())),
                        preferred_element_type=jnp.float32)
    for k in range(_BN // _PK):
        o_ref[k * 2048:(k + 1) * 2048, 0:_OUT] = (
            y[k * _PK:k * _PK + 2048])
        o_ref[k * 2048:(k + 1) * 2048, _OUT:2 * _OUT] = (
            y[k * _PK + 2048:(k + 1) * _PK])


def _tc_project_table(tableT, W, start, nsteps):
    return pl.pallas_call(
        _proj_body,
        grid=(nsteps,),
        in_specs=[
            pl.BlockSpec((_OUT, _BN), lambda j, s=start: (0, j + s)),
            pl.BlockSpec((_OUT, _OUT), lambda j: (0, 0)),
        ],
        out_specs=pl.BlockSpec((_BN // 2, 2 * _OUT), lambda j: (j, 0)),
        out_shape=jax.ShapeDtypeStruct((nsteps * _BN // 2, 2 * _OUT),
                                       jnp.float32),
    )(tableT, W)


def _make_sc_gather_body(rpw, cpw, p_base, slot_offset):
    def _sc_gather_body(hash_hbm, table_hbm, out_hbm,
                        hash_v, idx_v, rows0, rows1, sem0, sem1):
        wid = lax.axis_index("s") * _NC + lax.axis_index("c")
        i_base = wid * rpw
        pltpu.sync_copy(hash_hbm.at[pl.ds(i_base, rpw)], hash_v)

        def compute_idx(g, _):
            h = hash_v[pl.ds(g * 16, 16)]
            p = p_base + ((i_base + g * 16) >> 12)
            r = h + p * (_NUM_K + 1)
            slot = (lax.shift_left(lax.shift_right_logical(r, 12), 12)
                    | lax.shift_left(r & 2047, 1)
                    | (lax.shift_right_logical(r, 11) & 1))
            idx_v[g >> 3, pl.ds((g & 7) * 16, 16)] = slot - slot_offset
            return 0

        lax.fori_loop(0, rpw // 16, compute_idx, 0)

        def _dma(j, rows, sem):
            return pltpu.make_async_copy(table_hbm.at[idx_v.at[j]], rows, sem)

        def _writeout(j, rows):
            off = pl.multiple_of((wid * cpw + j) * _CH, _CH)
            pltpu.sync_copy(rows, out_hbm.at[pl.ds(off, _CH)])

        _dma(0, rows0, sem0).start()

        def pair(j2, _):
            j = 2 * j2
            _dma(j + 1, rows1, sem1).start()
            _dma(j, rows0, sem0).wait()
            _writeout(j, rows0)

            @pl.when(j2 + 1 < cpw // 2)
            def _():
                _dma(j + 2, rows0, sem0).start()

            _dma(j + 1, rows1, sem1).wait()
            _writeout(j + 1, rows1)
            return 0

        lax.fori_loop(0, cpw // 2, pair, 0)

    return _sc_gather_body


def _sc_gather(hashes_flat, proj_flat, p_base, slot_offset):
    n_slots = hashes_flat.shape[0]
    rpw = n_slots // _NW
    cpw = rpw // _CH
    mesh = plsc.VectorSubcoreMesh(core_axis_name="c", subcore_axis_name="s")
    return pl.kernel(
        _make_sc_gather_body(rpw, cpw, p_base, slot_offset),
        out_type=jax.ShapeDtypeStruct((n_slots, _OUT), jnp.float32),
        mesh=mesh,
        scratch_types=[
            pltpu.VMEM((rpw,), jnp.int32),
            pltpu.VMEM((cpw, _CH), jnp.int32),
            pltpu.VMEM((_CH, _OUT), jnp.float32),
            pltpu.VMEM((_CH, _OUT), jnp.float32),
            pltpu.SemaphoreType.DMA,
            pltpu.SemaphoreType.DMA,
        ],
        compiler_params=pltpu.CompilerParams(use_tc_tiling_on_sc=False),
    )(hashes_flat, proj_flat)


def _out_body(ga_ref, gb_ref, pe_ref, w_ref, b_ref, i_ref, o_ref):
    use_a = pl.program_id(0) < _PSPLIT
    g = jnp.where(use_a, ga_ref[0], gb_ref[0])
    eye = i_ref[...]
    pv = lax.dot_general(pe_ref[0], w_ref[...], (((1,), (1,)), ((), ())),
                         preferred_element_type=jnp.float32) + b_ref[...]
    x1 = g[:, 0:_OUT] + pv
    x2 = g[:, _OUT:2 * _OUT] + pv
    t1 = lax.dot_general(eye, x1, (((1,), (1,)), ((), ())),
                         preferred_element_type=jnp.float32)
    t2 = lax.dot_general(eye, x2, (((1,), (1,)), ((), ())),
                         preferred_element_type=jnp.float32)
    o_ref[0, :, 0:_B // 2] = t1
    o_ref[0, :, _B // 2:_B] = t2


@jax.jit
def _tc_output(gatheredA3, gatheredB3, pe3, W, b_col, eye):
    return pl.pallas_call(
        _out_body,
        grid=(_NUM_PARTS,),
        in_specs=[
            pl.BlockSpec((1, _B // 2, 2 * _OUT),
                         lambda p: (jnp.minimum(p, _PSPLIT - 1), 0, 0)),
            pl.BlockSpec((1, _B // 2, 2 * _OUT),
                         lambda p: (jnp.maximum(p - _PSPLIT, 0), 0, 0)),
            pl.BlockSpec((1, 1, _OUT), lambda p: (p, 0, 0)),
            pl.BlockSpec((_OUT, _OUT), lambda p: (0, 0)),
            pl.BlockSpec((1, _OUT), lambda p: (0, 0)),
            pl.BlockSpec((_OUT, _OUT), lambda p: (0, 0)),
        ],
        out_specs=pl.BlockSpec((1, _OUT, _B), lambda p: (p, 0, 0)),
        out_shape=jax.ShapeDtypeStruct((_NUM_PARTS, _OUT, _B), jnp.float32),
    )(gatheredA3, gatheredB3, pe3, W, b_col, eye)


def kernel(hashes, table, pe, W, b):
    tableT = table.T
    hashes_pm = (hashes.T.reshape(_NUM_PARTS, 2, _B // 2)
                 .transpose(0, 2, 1).reshape(_ROWS))
    nslots_a = _PSPLIT * _B
    proj2a = _tc_project_table(tableT, W, 0, _SP_A)
    gatheredA = _sc_gather(hashes_pm[:nslots_a],
                           proj2a.reshape(_SP_A * _BN, _OUT), 0, 0)
    proj2b = _tc_project_table(tableT, W, _SP_B0, _NB - _SP_B0)
    gatheredB = _sc_gather(hashes_pm[nslots_a:],
                           proj2b.reshape((_NB - _SP_B0) * _BN, _OUT),
                           _PSPLIT, _SP_B0 * _BN)
    gatheredA3 = gatheredA.reshape(_PSPLIT, _B // 2, 2 * _OUT)
    gatheredB3 = gatheredB.reshape(_NUM_PARTS - _PSPLIT, _B // 2, 2 * _OUT)
    pe3 = pe.reshape(_NUM_PARTS, 1, _OUT)
    out_pm = _tc_output(gatheredA3, gatheredB3, pe3, W, b.reshape(1, _OUT),
                        jnp.eye(_OUT, dtype=jnp.float32))
    return jnp.transpose(out_pm, (2, 0, 1))


# speedup vs baseline: 1.1377x; 1.1377x over previous
import jax
import jax.numpy as jnp
from jax import lax
from jax.experimental import pallas as pl
from jax.experimental.pallas import tpu as pltpu
from jax.experimental.pallas import tpu_sc as plsc

_NUM_PARTS = 100
_NUM_K = 9999
_OUT = 64
_B = 4096
_ROWS = _B * _NUM_PARTS
_V = (_NUM_K + 1) * _NUM_PARTS

_NC = 2
_NS = 16
_NW = _NC * _NS
_CH = 128
_RPW = _ROWS // _NW
_CPW = _RPW // _CH


_PK = 4096
_BN = 32768
_NB = (_V + _BN - 1) // _BN


def _proj_body(x_ref, w_ref, o_ref):
    y = lax.dot_general(x_ref[...], w_ref[...], (((0,), (1,)), ((), ())),
                        preferred_element_type=jnp.float32)
    for k in range(_BN // _PK):
        o_ref[k * 2048:(k + 1) * 2048, 0:_OUT] = (
            y[k * _PK:k * _PK + 2048])
        o_ref[k * 2048:(k + 1) * 2048, _OUT:2 * _OUT] = (
            y[k * _PK + 2048:(k + 1) * _PK])


@jax.jit
def _tc_project_table(tableT, W):
    return pl.pallas_call(
        _proj_body,
        grid=(_NB,),
        in_specs=[
            pl.BlockSpec((_OUT, _BN), lambda j: (0, j)),
            pl.BlockSpec((_OUT, _OUT), lambda j: (0, 0)),
        ],
        out_specs=pl.BlockSpec((_BN // 2, 2 * _OUT), lambda j: (j, 0)),
        out_shape=jax.ShapeDtypeStruct((_NB * _BN // 2, 2 * _OUT),
                                       jnp.float32),
    )(tableT, W)


def _sc_gather_body(hash_hbm, table_hbm, out_hbm,
                    hash_v, idx_v, rows0, rows1, sem0, sem1):
    wid = lax.axis_index("s") * _NC + lax.axis_index("c")
    i_base = wid * _RPW
    pltpu.sync_copy(hash_hbm.at[pl.ds(i_base, _RPW)], hash_v)

    def compute_idx(g, _):
        h = hash_v[pl.ds(g * 16, 16)]
        p = (i_base + g * 16) >> 12
        r = h + p * (_NUM_K + 1)
        slot = (lax.shift_left(lax.shift_right_logical(r, 12), 12)
                | lax.shift_left(r & 2047, 1)
                | (lax.shift_right_logical(r, 11) & 1))
        idx_v[g >> 3, pl.ds((g & 7) * 16, 16)] = slot
        return 0

    lax.fori_loop(0, _RPW // 16, compute_idx, 0)

    def _dma(j, rows, sem):
        return pltpu.make_async_copy(table_hbm.at[idx_v.at[j]], rows, sem)

    def _writeout(j, rows):
        off = pl.multiple_of((wid * _CPW + j) * _CH, _CH)
        pltpu.sync_copy(rows, out_hbm.at[pl.ds(off, _CH)])

    _dma(0, rows0, sem0).start()

    def pair(j2, _):
        j = 2 * j2
        _dma(j + 1, rows1, sem1).start()
        _dma(j, rows0, sem0).wait()
        _writeout(j, rows0)

        @pl.when(j2 + 1 < _CPW // 2)
        def _():
            _dma(j + 2, rows0, sem0).start()

        _dma(j + 1, rows1, sem1).wait()
        _writeout(j + 1, rows1)
        return 0

    lax.fori_loop(0, _CPW // 2, pair, 0)


@jax.jit
def _sc_gather(hashes_flat, proj_flat):
    mesh = plsc.VectorSubcoreMesh(core_axis_name="c", subcore_axis_name="s")
    return pl.kernel(
        _sc_gather_body,
        out_type=jax.ShapeDtypeStruct((_ROWS, _OUT), jnp.float32),
        mesh=mesh,
        scratch_types=[
            pltpu.VMEM((_RPW,), jnp.int32),
            pltpu.VMEM((_CPW, _CH), jnp.int32),
            pltpu.VMEM((_CH, _OUT), jnp.float32),
            pltpu.VMEM((_CH, _OUT), jnp.float32),
            pltpu.SemaphoreType.DMA,
            pltpu.SemaphoreType.DMA,
        ],
        compiler_params=pltpu.CompilerParams(use_tc_tiling_on_sc=False),
    )(hashes_flat, proj_flat)


def _out_body(g_ref, pe_ref, w_ref, b_ref, i_ref, o_ref):
    g = g_ref[0]
    eye = i_ref[...]
    pv = lax.dot_general(pe_ref[0], w_ref[...], (((1,), (1,)), ((), ())),
                         preferred_element_type=jnp.float32) + b_ref[...]
    x1 = g[:, 0:_OUT] + pv
    x2 = g[:, _OUT:2 * _OUT] + pv
    t1 = lax.dot_general(eye, x1, (((1,), (1,)), ((), ())),
                         preferred_element_type=jnp.float32)
    t2 = lax.dot_general(eye, x2, (((1,), (1,)), ((), ())),
                         preferred_element_type=jnp.float32)
    o_ref[0, :, 0:_B // 2] = t1
    o_ref[0, :, _B // 2:_B] = t2


@jax.jit
def _tc_output(gathered3, pe3, W, b_col, eye):
    return pl.pallas_call(
        _out_body,
        grid=(_NUM_PARTS,),
        in_specs=[
            pl.BlockSpec((1, _B // 2, 2 * _OUT), lambda p: (p, 0, 0)),
            pl.BlockSpec((1, 1, _OUT), lambda p: (p, 0, 0)),
            pl.BlockSpec((_OUT, _OUT), lambda p: (0, 0)),
            pl.BlockSpec((1, _OUT), lambda p: (0, 0)),
            pl.BlockSpec((_OUT, _OUT), lambda p: (0, 0)),
        ],
        out_specs=pl.BlockSpec((1, _OUT, _B), lambda p: (p, 0, 0)),
        out_shape=jax.ShapeDtypeStruct((_NUM_PARTS, _OUT, _B), jnp.float32),
    )(gathered3, pe3, W, b_col, eye)


def kernel(hashes, table, pe, W, b):
    tableT = table.T
    hashes_pm = (hashes.T.reshape(_NUM_PARTS, 2, _B // 2)
                 .transpose(0, 2, 1).reshape(_ROWS))
    proj2 = _tc_project_table(tableT, W)
    gathered = _sc_gather(hashes_pm, proj2.reshape(_NB * _BN, _OUT))
    gathered3 = gathered.reshape(_NUM_PARTS, _B // 2, 2 * _OUT)
    pe3 = pe.reshape(_NUM_PARTS, 1, _OUT)
    out_pm = _tc_output(gathered3, pe3, W, b.reshape(1, _OUT),
                        jnp.eye(_OUT, dtype=jnp.float32))
    return jnp.transpose(out_pm, (2, 0, 1))


# speedup vs baseline: 1.2021x; 1.0566x over previous
import jax
import jax.numpy as jnp
from jax import lax
from jax.experimental import pallas as pl
from jax.experimental.pallas import tpu as pltpu
from jax.experimental.pallas import tpu_sc as plsc

_NUM_PARTS = 100
_NUM_K = 9999
_OUT = 64
_B = 4096
_ROWS = _B * _NUM_PARTS
_V = (_NUM_K + 1) * _NUM_PARTS

_NC = 2
_NS = 16
_NW = _NC * _NS
_CH = 128
_RPW = _ROWS // _NW
_CPW = _RPW // _CH


_PK = 4096
_BN = 32768
_NB = (_V + _BN - 1) // _BN


def _proj_body(x_ref, w_ref, o_ref):
    y = lax.dot_general(x_ref[...], w_ref[...], (((0,), (1,)), ((), ())),
                        preferred_element_type=jnp.float32)
    for k in range(_BN // _PK):
        o_ref[k * 2048:(k + 1) * 2048, 0:_OUT] = (
            y[k * _PK:k * _PK + 2048])
        o_ref[k * 2048:(k + 1) * 2048, _OUT:2 * _OUT] = (
            y[k * _PK + 2048:(k + 1) * _PK])


@jax.jit
def _tc_project_table(tableT, W):
    return pl.pallas_call(
        _proj_body,
        grid=(_NB,),
        in_specs=[
            pl.BlockSpec((_OUT, _BN), lambda j: (0, j)),
            pl.BlockSpec((_OUT, _OUT), lambda j: (0, 0)),
        ],
        out_specs=pl.BlockSpec((_BN // 2, 2 * _OUT), lambda j: (j, 0)),
        out_shape=jax.ShapeDtypeStruct((_NB * _BN // 2, 2 * _OUT),
                                       jnp.float32),
    )(tableT, W)


def _sc_gather_body(hash_hbm, table_hbm, out_hbm,
                    hash_v, idx_v, rows0, rows1, sem0, sem1):
    wid = lax.axis_index("s") * _NC + lax.axis_index("c")
    i_base = wid * _RPW
    pltpu.sync_copy(hash_hbm.at[pl.ds(i_base, _RPW)], hash_v)

    def compute_idx(g, _):
        h = hash_v[pl.ds(g * 16, 16)]
        p = (i_base + g * 16) >> 12
        r = h + p * (_NUM_K + 1)
        slot = (lax.shift_left(lax.shift_right_logical(r, 12), 12)
                | lax.shift_left(r & 2047, 1)
                | (lax.shift_right_logical(r, 11) & 1))
        idx_v[g >> 3, pl.ds((g & 7) * 16, 16)] = slot
        return 0

    lax.fori_loop(0, _RPW // 16, compute_idx, 0)

    def _dma(j, rows, sem):
        return pltpu.make_async_copy(table_hbm.at[idx_v.at[j]], rows, sem)

    def _writeout(j, rows):
        off = pl.multiple_of((wid * _CPW + j) * _CH, _CH)
        pltpu.sync_copy(rows, out_hbm.at[pl.ds(off, _CH)])

    _dma(0, rows0, sem0).start()

    def pair(j2, _):
        j = 2 * j2
        _dma(j + 1, rows1, sem1).start()
        _dma(j, rows0, sem0).wait()
        _writeout(j, rows0)

        @pl.when(j2 + 1 < _CPW // 2)
        def _():
            _dma(j + 2, rows0, sem0).start()

        _dma(j + 1, rows1, sem1).wait()
        _writeout(j + 1, rows1)
        return 0

    lax.fori_loop(0, _CPW // 2, pair, 0)


@jax.jit
def _sc_gather(hashes_flat, proj_flat):
    mesh = plsc.VectorSubcoreMesh(core_axis_name="c", subcore_axis_name="s")
    return pl.kernel(
        _sc_gather_body,
        out_type=jax.ShapeDtypeStruct((_ROWS, _OUT), jnp.float32),
        mesh=mesh,
        scratch_types=[
            pltpu.VMEM((_RPW,), jnp.int32),
            pltpu.VMEM((_CPW, _CH), jnp.int32),
            pltpu.VMEM((_CH, _OUT), jnp.float32),
            pltpu.VMEM((_CH, _OUT), jnp.float32),
            pltpu.SemaphoreType.DMA,
            pltpu.SemaphoreType.DMA,
        ],
        compiler_params=pltpu.CompilerParams(use_tc_tiling_on_sc=False),
    )(hashes_flat, proj_flat)


_PP = 2


def _out_body(g_ref, pe_ref, w_ref, b_ref, i_ref, o_ref):
    eye = i_ref[...]
    for k in range(_PP):
        g = g_ref[k]
        pv = lax.dot_general(pe_ref[k], w_ref[...], (((1,), (1,)), ((), ())),
                             preferred_element_type=jnp.float32) + b_ref[...]
        x1 = g[:, 0:_OUT] + pv
        x2 = g[:, _OUT:2 * _OUT] + pv
        t1 = lax.dot_general(eye, x1, (((1,), (1,)), ((), ())),
                             preferred_element_type=jnp.float32)
        t2 = lax.dot_general(eye, x2, (((1,), (1,)), ((), ())),
                             preferred_element_type=jnp.float32)
        o_ref[k, :, 0:_B // 2] = t1
        o_ref[k, :, _B // 2:_B] = t2


@jax.jit
def _tc_output(gathered3, pe3, W, b_col, eye):
    return pl.pallas_call(
        _out_body,
        grid=(_NUM_PARTS // _PP,),
        in_specs=[
            pl.BlockSpec((_PP, _B // 2, 2 * _OUT), lambda p: (p, 0, 0)),
            pl.BlockSpec((_PP, 1, _OUT), lambda p: (p, 0, 0)),
            pl.BlockSpec((_OUT, _OUT), lambda p: (0, 0)),
            pl.BlockSpec((1, _OUT), lambda p: (0, 0)),
            pl.BlockSpec((_OUT, _OUT), lambda p: (0, 0)),
        ],
        out_specs=pl.BlockSpec((_PP, _OUT, _B), lambda p: (p, 0, 0)),
        out_shape=jax.ShapeDtypeStruct((_NUM_PARTS, _OUT, _B), jnp.float32),
    )(gathered3, pe3, W, b_col, eye)


def kernel(hashes, table, pe, W, b):
    tableT = table.T
    hashes_pm = (hashes.T.reshape(_NUM_PARTS, 2, _B // 2)
                 .transpose(0, 2, 1).reshape(_ROWS))
    proj2 = _tc_project_table(tableT, W)
    gathered = _sc_gather(hashes_pm, proj2.reshape(_NB * _BN, _OUT))
    gathered3 = gathered.reshape(_NUM_PARTS, _B // 2, 2 * _OUT)
    pe3 = pe.reshape(_NUM_PARTS, 1, _OUT)
    out_pm = _tc_output(gathered3, pe3, W, b.reshape(1, _OUT),
                        jnp.eye(_OUT, dtype=jnp.float32))
    return jnp.transpose(out_pm, (2, 0, 1))


# speedup vs baseline: 1.2343x; 1.0268x over previous
import jax
import jax.numpy as jnp
from jax import lax
from jax.experimental import pallas as pl
from jax.experimental.pallas import tpu as pltpu
from jax.experimental.pallas import tpu_sc as plsc

_NUM_PARTS = 100
_NUM_K = 9999
_OUT = 64
_B = 4096
_ROWS = _B * _NUM_PARTS
_V = (_NUM_K + 1) * _NUM_PARTS

_NC = 2
_NS = 16
_NW = _NC * _NS
_CH = 128
_RPW = _ROWS // _NW
_CPW = _RPW // _CH


_PK = 4096
_BN = 32768
_NB = (_V + _BN - 1) // _BN


def _proj_body(x_ref, w_ref, o_ref):
    y = lax.dot_general(x_ref[...], w_ref[...], (((0,), (1,)), ((), ())),
                        preferred_element_type=jnp.float32)
    for k in range(_BN // _PK):
        o_ref[k * 2048:(k + 1) * 2048, 0:_OUT] = (
            y[k * _PK:k * _PK + 2048])
        o_ref[k * 2048:(k + 1) * 2048, _OUT:2 * _OUT] = (
            y[k * _PK + 2048:(k + 1) * _PK])


@jax.jit
def _tc_project_table(tableT, W):
    return pl.pallas_call(
        _proj_body,
        grid=(_NB,),
        in_specs=[
            pl.BlockSpec((_OUT, _BN), lambda j: (0, j)),
            pl.BlockSpec((_OUT, _OUT), lambda j: (0, 0)),
        ],
        out_specs=pl.BlockSpec((_BN // 2, 2 * _OUT), lambda j: (j, 0)),
        out_shape=jax.ShapeDtypeStruct((_NB * _BN // 2, 2 * _OUT),
                                       jnp.float32),
    )(tableT, W)


def _sc_gather_body(hash_hbm, table_hbm, out_hbm,
                    hash_v, idx_v, rows0, rows1, sem0, sem1):
    wid = lax.axis_index("s") * _NC + lax.axis_index("c")
    i_base = wid * _RPW
    pltpu.sync_copy(hash_hbm.at[pl.ds(i_base, _RPW)], hash_v)

    def compute_idx(g, _):
        h = hash_v[pl.ds(g * 16, 16)]
        p = (i_base + g * 16) >> 12
        r = h + p * (_NUM_K + 1)
        slot = (lax.shift_left(lax.shift_right_logical(r, 12), 12)
                | lax.shift_left(r & 2047, 1)
                | (lax.shift_right_logical(r, 11) & 1))
        idx_v[g >> 3, pl.ds((g & 7) * 16, 16)] = slot
        return 0

    lax.fori_loop(0, _RPW // 16, compute_idx, 0)

    def _dma(j, rows, sem):
        return pltpu.make_async_copy(table_hbm.at[idx_v.at[j]], rows, sem)

    def _writeout(j, rows):
        off = pl.multiple_of((wid * _CPW + j) * _CH, _CH)
        pltpu.sync_copy(rows, out_hbm.at[pl.ds(off, _CH)])

    _dma(0, rows0, sem0).start()

    def pair(j2, _):
        j = 2 * j2
        _dma(j + 1, rows1, sem1).start()
        _dma(j, rows0, sem0).wait()
        _writeout(j, rows0)

        @pl.when(j2 + 1 < _CPW // 2)
        def _():
            _dma(j + 2, rows0, sem0).start()

        _dma(j + 1, rows1, sem1).wait()
        _writeout(j + 1, rows1)
        return 0

    lax.fori_loop(0, _CPW // 2, pair, 0)


@jax.jit
def _sc_gather(hashes_flat, proj_flat):
    mesh = plsc.VectorSubcoreMesh(core_axis_name="c", subcore_axis_name="s")
    return pl.kernel(
        _sc_gather_body,
        out_type=jax.ShapeDtypeStruct((_ROWS, _OUT), jnp.float32),
        mesh=mesh,
        scratch_types=[
            pltpu.VMEM((_RPW,), jnp.int32),
            pltpu.VMEM((_CPW, _CH), jnp.int32),
            pltpu.VMEM((_CH, _OUT), jnp.float32),
            pltpu.VMEM((_CH, _OUT), jnp.float32),
            pltpu.SemaphoreType.DMA,
            pltpu.SemaphoreType.DMA,
        ],
        compiler_params=pltpu.CompilerParams(use_tc_tiling_on_sc=False),
    )(hashes_flat, proj_flat)


_PP = 4


def _out_body(g_ref, pe_ref, w_ref, b_ref, i_ref, o_ref):
    eye = i_ref[...]
    for k in range(_PP):
        g = g_ref[k]
        pv = lax.dot_general(pe_ref[k], w_ref[...], (((1,), (1,)), ((), ())),
                             preferred_element_type=jnp.float32) + b_ref[...]
        x1 = g[:, 0:_OUT] + pv
        x2 = g[:, _OUT:2 * _OUT] + pv
        t1 = lax.dot_general(eye, x1, (((1,), (1,)), ((), ())),
                             preferred_element_type=jnp.float32)
        t2 = lax.dot_general(eye, x2, (((1,), (1,)), ((), ())),
                             preferred_element_type=jnp.float32)
        o_ref[k, :, 0:_B // 2] = t1
        o_ref[k, :, _B // 2:_B] = t2


@jax.jit
def _tc_output(gathered3, pe3, W, b_col, eye):
    return pl.pallas_call(
        _out_body,
        grid=(_NUM_PARTS // _PP,),
        in_specs=[
            pl.BlockSpec((_PP, _B // 2, 2 * _OUT), lambda p: (p, 0, 0)),
            pl.BlockSpec((_PP, 1, _OUT), lambda p: (p, 0, 0)),
            pl.BlockSpec((_OUT, _OUT), lambda p: (0, 0)),
            pl.BlockSpec((1, _OUT), lambda p: (0, 0)),
            pl.BlockSpec((_OUT, _OUT), lambda p: (0, 0)),
        ],
        out_specs=pl.BlockSpec((_PP, _OUT, _B), lambda p: (p, 0, 0)),
        out_shape=jax.ShapeDtypeStruct((_NUM_PARTS, _OUT, _B), jnp.float32),
    )(gathered3, pe3, W, b_col, eye)


def kernel(hashes, table, pe, W, b):
    tableT = table.T
    hashes_pm = (hashes.T.reshape(_NUM_PARTS, 2, _B // 2)
                 .transpose(0, 2, 1).reshape(_ROWS))
    proj2 = _tc_project_table(tableT, W)
    gathered = _sc_gather(hashes_pm, proj2.reshape(_NB * _BN, _OUT))
    gathered3 = gathered.reshape(_NUM_PARTS, _B // 2, 2 * _OUT)
    pe3 = pe.reshape(_NUM_PARTS, 1, _OUT)
    out_pm = _tc_output(gathered3, pe3, W, b.reshape(1, _OUT),
                        jnp.eye(_OUT, dtype=jnp.float32))
    return jnp.transpose(out_pm, (2, 0, 1))


# speedup vs baseline: 1.2492x; 1.0120x over previous
import jax
import jax.numpy as jnp
from jax import lax
from jax.experimental import pallas as pl
from jax.experimental.pallas import tpu as pltpu
from jax.experimental.pallas import tpu_sc as plsc

_NUM_PARTS = 100
_NUM_K = 9999
_OUT = 64
_B = 4096
_ROWS = _B * _NUM_PARTS
_V = (_NUM_K + 1) * _NUM_PARTS

_NC = 2
_NS = 16
_NW = _NC * _NS
_CH = 128
_RPW = _ROWS // _NW
_CPW = _RPW // _CH


_PK = 4096
_BN = 32768
_NB = (_V + _BN - 1) // _BN


def _proj_body(x_ref, w_ref, o_ref):
    y = lax.dot_general(x_ref[...], w_ref[...], (((0,), (1,)), ((), ())),
                        preferred_element_type=jnp.float32)
    for k in range(_BN // _PK):
        o_ref[k * 2048:(k + 1) * 2048, 0:_OUT] = (
            y[k * _PK:k * _PK + 2048])
        o_ref[k * 2048:(k + 1) * 2048, _OUT:2 * _OUT] = (
            y[k * _PK + 2048:(k + 1) * _PK])


@jax.jit
def _tc_project_table(tableT, W):
    return pl.pallas_call(
        _proj_body,
        grid=(_NB,),
        in_specs=[
            pl.BlockSpec((_OUT, _BN), lambda j: (0, j)),
            pl.BlockSpec((_OUT, _OUT), lambda j: (0, 0)),
        ],
        out_specs=pl.BlockSpec((_BN // 2, 2 * _OUT), lambda j: (j, 0)),
        out_shape=jax.ShapeDtypeStruct((_NB * _BN // 2, 2 * _OUT),
                                       jnp.float32),
    )(tableT, W)


def _sc_gather_body(hash_hbm, table_hbm, out_hbm,
                    hash_v, idx_v, rows0, rows1, sem0, sem1):
    wid = lax.axis_index("s") * _NC + lax.axis_index("c")
    i_base = wid * _RPW
    pltpu.sync_copy(hash_hbm.at[pl.ds(i_base, _RPW)], hash_v)

    def compute_idx(g, _):
        h = hash_v[pl.ds(g * 16, 16)]
        p = (i_base + g * 16) >> 12
        r = h + p * (_NUM_K + 1)
        slot = (lax.shift_left(lax.shift_right_logical(r, 12), 12)
                | lax.shift_left(r & 2047, 1)
                | (lax.shift_right_logical(r, 11) & 1))
        idx_v[g >> 3, pl.ds((g & 7) * 16, 16)] = slot
        return 0

    lax.fori_loop(0, _RPW // 16, compute_idx, 0)

    def _dma(j, rows, sem):
        return pltpu.make_async_copy(table_hbm.at[idx_v.at[j]], rows, sem)

    def _writeout(j, rows):
        off = pl.multiple_of((wid * _CPW + j) * _CH, _CH)
        pltpu.sync_copy(rows, out_hbm.at[pl.ds(off, _CH)])

    _dma(0, rows0, sem0).start()

    def pair(j2, _):
        j = 2 * j2
        _dma(j + 1, rows1, sem1).start()
        _dma(j, rows0, sem0).wait()
        _writeout(j, rows0)

        @pl.when(j2 + 1 < _CPW // 2)
        def _():
            _dma(j + 2, rows0, sem0).start()

        _dma(j + 1, rows1, sem1).wait()
        _writeout(j + 1, rows1)
        return 0

    lax.fori_loop(0, _CPW // 2, pair, 0)


@jax.jit
def _sc_gather(hashes_flat, proj_flat):
    mesh = plsc.VectorSubcoreMesh(core_axis_name="c", subcore_axis_name="s")
    return pl.kernel(
        _sc_gather_body,
        out_type=jax.ShapeDtypeStruct((_ROWS, _OUT), jnp.float32),
        mesh=mesh,
        scratch_types=[
            pltpu.VMEM((_RPW,), jnp.int32),
            pltpu.VMEM((_CPW, _CH), jnp.int32),
            pltpu.VMEM((_CH, _OUT), jnp.float32),
            pltpu.VMEM((_CH, _OUT), jnp.float32),
            pltpu.SemaphoreType.DMA,
            pltpu.SemaphoreType.DMA,
        ],
        compiler_params=pltpu.CompilerParams(use_tc_tiling_on_sc=False),
    )(hashes_flat, proj_flat)


_PP = 10


def _out_body(g_ref, pe_ref, w_ref, b_ref, i_ref, o_ref):
    eye = i_ref[...]
    for k in range(_PP):
        g = g_ref[k]
        pv = lax.dot_general(pe_ref[k], w_ref[...], (((1,), (1,)), ((), ())),
                             preferred_element_type=jnp.float32) + b_ref[...]
        x1 = g[:, 0:_OUT] + pv
        x2 = g[:, _OUT:2 * _OUT] + pv
        t1 = lax.dot_general(eye, x1, (((1,), (1,)), ((), ())),
                             preferred_element_type=jnp.float32)
        t2 = lax.dot_general(eye, x2, (((1,), (1,)), ((), ())),
                             preferred_element_type=jnp.float32)
        o_ref[k, :, 0:_B // 2] = t1
        o_ref[k, :, _B // 2:_B] = t2


@jax.jit
def _tc_output(gathered3, pe3, W, b_col, eye):
    return pl.pallas_call(
        _out_body,
        grid=(_NUM_PARTS // _PP,),
        in_specs=[
            pl.BlockSpec((_PP, _B // 2, 2 * _OUT), lambda p: (p, 0, 0)),
            pl.BlockSpec((_PP, 1, _OUT), lambda p: (p, 0, 0)),
            pl.BlockSpec((_OUT, _OUT), lambda p: (0, 0)),
            pl.BlockSpec((1, _OUT), lambda p: (0, 0)),
            pl.BlockSpec((_OUT, _OUT), lambda p: (0, 0)),
        ],
        out_specs=pl.BlockSpec((_PP, _OUT, _B), lambda p: (p, 0, 0)),
        out_shape=jax.ShapeDtypeStruct((_NUM_PARTS, _OUT, _B), jnp.float32),
    )(gathered3, pe3, W, b_col, eye)


def kernel(hashes, table, pe, W, b):
    tableT = table.T
    hashes_pm = (hashes.T.reshape(_NUM_PARTS, 2, _B // 2)
                 .transpose(0, 2, 1).reshape(_ROWS))
    proj2 = _tc_project_table(tableT, W)
    gathered = _sc_gather(hashes_pm, proj2.reshape(_NB * _BN, _OUT))
    gathered3 = gathered.reshape(_NUM_PARTS, _B // 2, 2 * _OUT)
    pe3 = pe.reshape(_NUM_PARTS, 1, _OUT)
    out_pm = _tc_output(gathered3, pe3, W, b.reshape(1, _OUT),
                        jnp.eye(_OUT, dtype=jnp.float32))
    return jnp.transpose(out_pm, (2, 0, 1))
